# Initial kernel scaffold; baseline (speedup 1.0000x reference)
import functools
import math

import jax
import jax.numpy as jnp
from jax import lax
from jax.experimental import pallas as pl
from jax.experimental.pallas import tpu as pltpu
from jax.experimental.pallas import tpu_sc as plsc

ALPHA = 0.2
BN_EPS = 1e-5

NC = 2
NS = 16
NW = NC * NS
WIN = 128


def _sc_mesh():
    return plsc.VectorSubcoreMesh(core_axis_name="c", subcore_axis_name="s")


def _make_deg_kernel(n_pad, nwin, rows_pt):
    @functools.partial(
        pl.kernel,
        out_type=jax.ShapeDtypeStruct((NC, n_pad), jnp.float32),
        mesh=_sc_mesh(),
        scratch_types=[
            pltpu.VMEM((nwin, WIN), jnp.int32),
            pltpu.VMEM((WIN,), jnp.float32),
            pltpu.VMEM_SHARED((n_pad,), jnp.float32),
        ],
    )
    def deg_kernel(dst_hbm, zeros_hbm, out_hbm, idx_v, ones_v, deg_sp):
        cid = lax.axis_index("c")
        sid = lax.axis_index("s")
        wid = cid * NS + sid
        rbase = sid * rows_pt
        pltpu.sync_copy(zeros_hbm.at[pl.ds(rbase, rows_pt)],
                        deg_sp.at[pl.ds(rbase, rows_pt)])
        for i in range(WIN // 16):
            ones_v[pl.ds(i * 16, 16)] = jnp.ones((16,), jnp.float32)
        pltpu.sync_copy(dst_hbm.at[wid], idx_v)
        plsc.subcore_barrier()

        def body(j, carry):
            pltpu.sync_copy(ones_v, deg_sp.at[idx_v.at[j]], add=True)
            return carry

        lax.fori_loop(0, nwin, body, 0)
        plsc.subcore_barrier()
        pltpu.sync_copy(deg_sp.at[pl.ds(rbase, rows_pt)],
                        out_hbm.at[cid, pl.ds(rbase, rows_pt)])

    return deg_kernel


def _make_agg_kernel(n_pad, d, nwin, rows_pt):
    @functools.partial(
        pl.kernel,
        out_type=jax.ShapeDtypeStruct((NC, n_pad, d), jnp.float32),
        mesh=_sc_mesh(),
        scratch_types=[
            pltpu.VMEM((nwin, WIN), jnp.int32),
            pltpu.VMEM((nwin, WIN), jnp.int32),
            pltpu.VMEM((WIN, d), jnp.float32),
            pltpu.VMEM_SHARED((n_pad, d), jnp.float32),
            pltpu.SemaphoreType.DMA,
        ],
    )
    def agg_kernel(src_hbm, dst_hbm, xs_hbm, zeros_hbm, out_hbm,
                   si_v, di_v, rows_v, agg_sp, gsem):
        cid = lax.axis_index("c")
        sid = lax.axis_index("s")
        wid = cid * NS + sid
        rbase = sid * rows_pt
        pltpu.sync_copy(zeros_hbm.at[pl.ds(rbase, rows_pt)],
                        agg_sp.at[pl.ds(rbase, rows_pt)])
        pltpu.sync_copy(src_hbm.at[wid], si_v)
        pltpu.sync_copy(dst_hbm.at[wid], di_v)
        plsc.subcore_barrier()

        def body(j, carry):
            pltpu.async_copy(xs_hbm.at[si_v.at[j]], rows_v, gsem).wait()
            pltpu.sync_copy(rows_v, agg_sp.at[di_v.at[j]], add=True)
            return carry

        lax.fori_loop(0, nwin, body, 0)
        plsc.subcore_barrier()
        pltpu.sync_copy(agg_sp.at[pl.ds(rbase, rows_pt)],
                        out_hbm.at[cid, pl.ds(rbase, rows_pt)])

    return agg_kernel


def _norm_xs_body(d0_ref, d1_ref, h_ref, norm_ref, xs_ref):
    deg = jnp.maximum(d0_ref[...] + d1_ref[...], 1.0)
    nrm = lax.rsqrt(deg)
    norm_ref[...] = nrm
    xs_ref[...] = h_ref[...] * nrm


def _layer_body(p0_ref, p1_ref, h_ref, nrm_ref, w_ref, g_ref, b_ref,
                out_ref, *, beta, bn_scale):
    nrm = nrm_ref[...]
    smoothed = (p0_ref[...] + p1_ref[...]) * nrm
    feat = (1.0 - ALPHA) * smoothed + ALPHA * h_ref[...]
    z = (1.0 - beta) * feat + beta * jnp.dot(
        feat, w_ref[...], preferred_element_type=jnp.float32)
    y = jnp.maximum(z * bn_scale * g_ref[...] + b_ref[...], 0.0)
    out_ref[...] = y * nrm


def _final_body(p0_ref, p1_ref, h_ref, nrm_ref, w_ref, out_ref, *, beta):
    smoothed = (p0_ref[...] + p1_ref[...]) * nrm_ref[...]
    feat = (1.0 - ALPHA) * smoothed + ALPHA * h_ref[...]
    out_ref[...] = (1.0 - beta) * feat + beta * jnp.dot(
        feat, w_ref[...], preferred_element_type=jnp.float32)


def kernel(graph, h, W1, W2, W3, gamma1, beta1, gamma2, beta2):
    n, d = h.shape
    e = graph.shape[1]

    n_pad = ((n + NS * 8 - 1) // (NS * 8)) * (NS * 8)
    if n_pad == n:
        n_pad += NS * 8
    pad_rows = n_pad - n
    rows_pt = n_pad // NS
    e_pad = ((e + NW * WIN - 1) // (NW * WIN)) * (NW * WIN)
    nwin = e_pad // (NW * WIN)

    src = graph[0].astype(jnp.int32)
    dst = graph[1].astype(jnp.int32)
    pad_idx = n + (jnp.arange(e_pad - e, dtype=jnp.int32) % pad_rows)
    src_w = jnp.concatenate([src, pad_idx]).reshape(NW, nwin, WIN)
    dst_w = jnp.concatenate([dst, pad_idx]).reshape(NW, nwin, WIN)
    h_pad = jnp.pad(h, ((0, pad_rows), (0, 0)))
    zeros_nd = jnp.zeros((n_pad, d), jnp.float32)
    zeros_n = jnp.zeros((n_pad,), jnp.float32)

    deg_kernel = _make_deg_kernel(n_pad, nwin, rows_pt)
    agg_kernel = _make_agg_kernel(n_pad, d, nwin, rows_pt)

    dparts = deg_kernel(dst_w, zeros_n)

    blk = n_pad // 8
    grid = (n_pad // blk,)
    row_spec = pl.BlockSpec((blk, d), lambda i: (i, 0))
    col_spec = pl.BlockSpec((blk, 1), lambda i: (i, 0))
    w_spec = pl.BlockSpec((d, d), lambda i: (0, 0))
    vec_spec = pl.BlockSpec((1, d), lambda i: (0, 0))

    norm2d, xs = pl.pallas_call(
        _norm_xs_body,
        grid=grid,
        in_specs=[col_spec, col_spec, row_spec],
        out_specs=[col_spec, row_spec],
        out_shape=[jax.ShapeDtypeStruct((n_pad, 1), jnp.float32),
                   jax.ShapeDtypeStruct((n_pad, d), jnp.float32)],
    )(dparts[0].reshape(n_pad, 1), dparts[1].reshape(n_pad, 1), h_pad)

    beta_l = [math.log(1.0 / l + 1.0) for l in (1, 2, 3)]
    bn_scale = 1.0 / math.sqrt(1.0 + BN_EPS)

    for li, (w, g, b) in enumerate(((W1, gamma1, beta1),
                                    (W2, gamma2, beta2))):
        parts = agg_kernel(src_w, dst_w, xs, zeros_nd)
        xs = pl.pallas_call(
            functools.partial(_layer_body, beta=beta_l[li],
                              bn_scale=bn_scale),
            grid=grid,
            in_specs=[row_spec, row_spec, row_spec, col_spec, w_spec,
                      vec_spec, vec_spec],
            out_specs=row_spec,
            out_shape=jax.ShapeDtypeStruct((n_pad, d), jnp.float32),
        )(parts[0], parts[1], h_pad, norm2d, w,
          g.reshape(1, d), b.reshape(1, d))

    parts = agg_kernel(src_w, dst_w, xs, zeros_nd)
    out = pl.pallas_call(
        functools.partial(_final_body, beta=beta_l[2]),
        grid=grid,
        in_specs=[row_spec, row_spec, row_spec, col_spec, w_spec],
        out_specs=row_spec,
        out_shape=jax.ShapeDtypeStruct((n_pad, d), jnp.float32),
    )(parts[0], parts[1], h_pad, norm2d, W3)

    return out[:n]


# speedup vs baseline: 7.3216x; 7.3216x over previous
import functools
import math

import jax
import jax.numpy as jnp
from jax import lax
from jax.experimental import pallas as pl
from jax.experimental.pallas import tpu as pltpu
from jax.experimental.pallas import tpu_sc as plsc

ALPHA = 0.2
BN_EPS = 1e-5

NC = 2
NS = 16
NW = NC * NS
WIN = 128


def _sc_mesh():
    return plsc.VectorSubcoreMesh(core_axis_name="c", subcore_axis_name="s")


def _make_deg_kernel(n_pad, nwin, rows_pt):
    @functools.partial(
        pl.kernel,
        out_type=jax.ShapeDtypeStruct((NC * n_pad,), jnp.float32),
        mesh=_sc_mesh(),
        scratch_types=[
            pltpu.VMEM((nwin, WIN), jnp.int32),
            pltpu.VMEM((WIN,), jnp.float32),
            pltpu.VMEM((rows_pt,), jnp.float32),
            pltpu.VMEM_SHARED((n_pad,), jnp.float32),
        ],
    )
    def deg_kernel(dst_hbm, zeros_hbm, out_hbm, idx_v, ones_v, bounce_v,
                   deg_sp):
        cid = lax.axis_index("c")
        sid = lax.axis_index("s")
        wid = cid * NS + sid
        rbase = pl.multiple_of(sid * rows_pt, 8)
        obase = pl.multiple_of(cid * n_pad + rbase, 8)
        pltpu.sync_copy(zeros_hbm, bounce_v)
        pltpu.sync_copy(bounce_v, deg_sp.at[pl.ds(rbase, rows_pt)])
        for i in range(WIN // 16):
            ones_v[pl.ds(i * 16, 16)] = jnp.ones((16,), jnp.float32)
        pltpu.sync_copy(dst_hbm.at[wid], idx_v)
        plsc.subcore_barrier()

        def body(j, carry):
            pltpu.sync_copy(ones_v, deg_sp.at[idx_v.at[j]], add=True)
            return carry

        lax.fori_loop(0, nwin, body, 0)
        plsc.subcore_barrier()
        pltpu.sync_copy(deg_sp.at[pl.ds(rbase, rows_pt)], bounce_v)
        pltpu.sync_copy(bounce_v, out_hbm.at[pl.ds(obase, rows_pt)])

    return deg_kernel


def _make_agg_kernel(n_pad, d, nwin, rows_pt):
    @functools.partial(
        pl.kernel,
        out_type=jax.ShapeDtypeStruct((NC * n_pad, d), jnp.float32),
        mesh=_sc_mesh(),
        scratch_types=[
            pltpu.VMEM((nwin, WIN), jnp.int32),
            pltpu.VMEM((nwin, WIN), jnp.int32),
            pltpu.VMEM((WIN, d), jnp.float32),
            pltpu.VMEM_SHARED((n_pad, d), jnp.float32),
            pltpu.SemaphoreType.DMA,
        ],
    )
    def agg_kernel(src_hbm, dst_hbm, xs_hbm, zeros_hbm, out_hbm,
                   si_v, di_v, rows_v, agg_sp, gsem):
        cid = lax.axis_index("c")
        sid = lax.axis_index("s")
        wid = cid * NS + sid
        rbase = sid * rows_pt
        obase = cid * n_pad + rbase
        pltpu.sync_copy(zeros_hbm, rows_v)
        for off in range(0, rows_pt, WIN):
            sz = min(WIN, rows_pt - off)
            pltpu.sync_copy(
                rows_v.at[pl.ds(0, sz)],
                agg_sp.at[pl.ds(pl.multiple_of(rbase + off, 8), sz)])
        pltpu.sync_copy(src_hbm.at[wid], si_v)
        pltpu.sync_copy(dst_hbm.at[wid], di_v)
        plsc.subcore_barrier()

        def body(j, carry):
            pltpu.async_copy(xs_hbm.at[si_v.at[j]], rows_v, gsem).wait()
            pltpu.sync_copy(rows_v, agg_sp.at[di_v.at[j]], add=True)
            return carry

        lax.fori_loop(0, nwin, body, 0)
        plsc.subcore_barrier()
        for off in range(0, rows_pt, WIN):
            sz = min(WIN, rows_pt - off)
            pltpu.sync_copy(
                agg_sp.at[pl.ds(pl.multiple_of(rbase + off, 8), sz)],
                rows_v.at[pl.ds(0, sz)])
            pltpu.sync_copy(
                rows_v.at[pl.ds(0, sz)],
                out_hbm.at[pl.ds(pl.multiple_of(obase + off, 8), sz)])

    return agg_kernel


def _norm_xs_body(d0_ref, d1_ref, h_ref, norm_ref, xs_ref):
    deg = jnp.maximum(d0_ref[...] + d1_ref[...], 1.0)
    nrm = lax.rsqrt(deg)
    norm_ref[...] = nrm
    xs_ref[...] = h_ref[...] * nrm


def _layer_body(p0_ref, p1_ref, h_ref, nrm_ref, w_ref, g_ref, b_ref,
                out_ref, *, beta, bn_scale):
    nrm = nrm_ref[...]
    smoothed = (p0_ref[...] + p1_ref[...]) * nrm
    feat = (1.0 - ALPHA) * smoothed + ALPHA * h_ref[...]
    z = (1.0 - beta) * feat + beta * jnp.dot(
        feat, w_ref[...], preferred_element_type=jnp.float32)
    y = jnp.maximum(z * bn_scale * g_ref[...] + b_ref[...], 0.0)
    out_ref[...] = y * nrm


def _final_body(p0_ref, p1_ref, h_ref, nrm_ref, w_ref, out_ref, *, beta):
    smoothed = (p0_ref[...] + p1_ref[...]) * nrm_ref[...]
    feat = (1.0 - ALPHA) * smoothed + ALPHA * h_ref[...]
    out_ref[...] = (1.0 - beta) * feat + beta * jnp.dot(
        feat, w_ref[...], preferred_element_type=jnp.float32)


def kernel(graph, h, W1, W2, W3, gamma1, beta1, gamma2, beta2):
    n, d = h.shape
    e = graph.shape[1]

    n_pad = ((n + NS * 8 - 1) // (NS * 8)) * (NS * 8)
    if n_pad == n:
        n_pad += NS * 8
    pad_rows = n_pad - n
    rows_pt = n_pad // NS
    e_pad = ((e + NW * WIN - 1) // (NW * WIN)) * (NW * WIN)
    nwin = e_pad // (NW * WIN)

    src = graph[0].astype(jnp.int32)
    dst = graph[1].astype(jnp.int32)
    pad_idx = n + (jnp.arange(e_pad - e, dtype=jnp.int32) % pad_rows)
    src_w = jnp.concatenate([src, pad_idx]).reshape(NW, nwin, WIN)
    dst_w = jnp.concatenate([dst, pad_idx]).reshape(NW, nwin, WIN)
    h_pad = jnp.pad(h, ((0, pad_rows), (0, 0)))
    zeros_nd = jnp.zeros((WIN, d), jnp.float32)
    zeros_n = jnp.zeros((rows_pt,), jnp.float32)

    deg_kernel = _make_deg_kernel(n_pad, nwin, rows_pt)
    agg_kernel = _make_agg_kernel(n_pad, d, nwin, rows_pt)

    dparts = deg_kernel(dst_w, zeros_n)

    blk = n_pad // 8
    grid = (n_pad // blk,)
    row_spec = pl.BlockSpec((blk, d), lambda i: (i, 0))
    col_spec = pl.BlockSpec((blk, 1), lambda i: (i, 0))
    w_spec = pl.BlockSpec((d, d), lambda i: (0, 0))
    vec_spec = pl.BlockSpec((1, d), lambda i: (0, 0))

    norm2d, xs = pl.pallas_call(
        _norm_xs_body,
        grid=grid,
        in_specs=[col_spec, col_spec, row_spec],
        out_specs=[col_spec, row_spec],
        out_shape=[jax.ShapeDtypeStruct((n_pad, 1), jnp.float32),
                   jax.ShapeDtypeStruct((n_pad, d), jnp.float32)],
    )(dparts[:n_pad].reshape(n_pad, 1), dparts[n_pad:].reshape(n_pad, 1),
      h_pad)

    beta_l = [math.log(1.0 / l + 1.0) for l in (1, 2, 3)]
    bn_scale = 1.0 / math.sqrt(1.0 + BN_EPS)

    for li, (w, g, b) in enumerate(((W1, gamma1, beta1),
                                    (W2, gamma2, beta2))):
        parts = agg_kernel(src_w, dst_w, xs, zeros_nd)
        xs = pl.pallas_call(
            functools.partial(_layer_body, beta=beta_l[li],
                              bn_scale=bn_scale),
            grid=grid,
            in_specs=[row_spec, row_spec, row_spec, col_spec, w_spec,
                      vec_spec, vec_spec],
            out_specs=row_spec,
            out_shape=jax.ShapeDtypeStruct((n_pad, d), jnp.float32),
        )(parts[:n_pad], parts[n_pad:], h_pad, norm2d, w,
          g.reshape(1, d), b.reshape(1, d))

    parts = agg_kernel(src_w, dst_w, xs, zeros_nd)
    out = pl.pallas_call(
        functools.partial(_final_body, beta=beta_l[2]),
        grid=grid,
        in_specs=[row_spec, row_spec, row_spec, col_spec, w_spec],
        out_specs=row_spec,
        out_shape=jax.ShapeDtypeStruct((n_pad, d), jnp.float32),
    )(parts[:n_pad], parts[n_pad:], h_pad, norm2d, W3)

    return out[:n]


# speedup vs baseline: 9.1284x; 1.2468x over previous
import functools
import math

import jax
import jax.numpy as jnp
from jax import lax
from jax.experimental import pallas as pl
from jax.experimental.pallas import tpu as pltpu
from jax.experimental.pallas import tpu_sc as plsc

ALPHA = 0.2
BN_EPS = 1e-5

NC = 2
NS = 16
NW = NC * NS
WIN = 128
CH = 8


def _sc_mesh():
    return plsc.VectorSubcoreMesh(core_axis_name="c", subcore_axis_name="s")


def _make_deg_kernel(n_pad, nwin, rows_pt):
    @functools.partial(
        pl.kernel,
        out_type=jax.ShapeDtypeStruct((NC * n_pad,), jnp.float32),
        mesh=_sc_mesh(),
        scratch_types=[
            pltpu.VMEM((nwin, WIN), jnp.int32),
            pltpu.VMEM((WIN,), jnp.float32),
            pltpu.VMEM((rows_pt,), jnp.float32),
            pltpu.VMEM_SHARED((n_pad,), jnp.float32),
        ],
    )
    def deg_kernel(dst_hbm, zeros_hbm, out_hbm, idx_v, ones_v, bounce_v,
                   deg_sp):
        cid = lax.axis_index("c")
        sid = lax.axis_index("s")
        wid = cid * NS + sid
        rbase = pl.multiple_of(sid * rows_pt, 8)
        obase = pl.multiple_of(cid * n_pad + rbase, 8)
        pltpu.sync_copy(zeros_hbm, bounce_v)
        pltpu.sync_copy(bounce_v, deg_sp.at[pl.ds(rbase, rows_pt)])
        for i in range(WIN // 16):
            ones_v[pl.ds(i * 16, 16)] = jnp.ones((16,), jnp.float32)
        pltpu.sync_copy(dst_hbm.at[wid], idx_v)
        plsc.subcore_barrier()

        def body(j, carry):
            pltpu.sync_copy(ones_v, deg_sp.at[idx_v.at[j]], add=True)
            return carry

        lax.fori_loop(0, nwin, body, 0)
        plsc.subcore_barrier()
        pltpu.sync_copy(deg_sp.at[pl.ds(rbase, rows_pt)], bounce_v)
        pltpu.sync_copy(bounce_v, out_hbm.at[pl.ds(obase, rows_pt)])

    return deg_kernel


def _make_agg_kernel(n_pad, d, nwin, rows_pt):
    @functools.partial(
        pl.kernel,
        out_type=jax.ShapeDtypeStruct((NC * n_pad, d), jnp.float32),
        mesh=_sc_mesh(),
        scratch_types=[
            pltpu.VMEM((2, CH, WIN), jnp.int32),
            pltpu.VMEM((2, CH, WIN), jnp.int32),
            pltpu.VMEM((2, WIN, d), jnp.float32),
            pltpu.VMEM_SHARED((n_pad, d), jnp.float32),
            pltpu.SemaphoreType.DMA,
            pltpu.SemaphoreType.DMA,
        ],
    )
    def agg_kernel(src_hbm, dst_hbm, xs_hbm, zeros_hbm, out_hbm,
                   sic, dic, rows_v, agg_sp, gsem, isem):
        nchunks = nwin // CH
        cid = lax.axis_index("c")
        sid = lax.axis_index("s")
        wid = cid * NS + sid
        rbase = sid * rows_pt
        obase = cid * n_pad + rbase
        pltpu.sync_copy(zeros_hbm, rows_v.at[0])
        for off in range(0, rows_pt, WIN):
            sz = min(WIN, rows_pt - off)
            pltpu.sync_copy(
                rows_v.at[0, pl.ds(0, sz)],
                agg_sp.at[pl.ds(pl.multiple_of(rbase + off, 8), sz)])
        pltpu.sync_copy(src_hbm.at[wid, pl.ds(0, CH)], sic.at[0])
        pltpu.sync_copy(dst_hbm.at[wid, pl.ds(0, CH)], dic.at[0])
        pltpu.async_copy(src_hbm.at[wid, pl.ds(CH, CH)], sic.at[1], isem)
        pltpu.async_copy(dst_hbm.at[wid, pl.ds(CH, CH)], dic.at[1], isem)
        plsc.subcore_barrier()

        pltpu.async_copy(xs_hbm.at[sic.at[0, 0]], rows_v.at[0], gsem)

        def chunk_body(c2, carry):
            for p in range(2):
                c = c2 * 2 + p
                for b in range(CH):
                    buf = b % 2
                    pltpu.make_async_copy(
                        xs_hbm.at[sic.at[p, b]], rows_v.at[buf],
                        gsem).wait()
                    if b + 1 < CH:
                        pltpu.async_copy(
                            xs_hbm.at[sic.at[p, b + 1]],
                            rows_v.at[1 - buf], gsem)
                        pltpu.sync_copy(rows_v.at[buf],
                                        agg_sp.at[dic.at[p, b]], add=True)
                    else:
                        pltpu.make_async_copy(
                            src_hbm.at[wid, pl.ds(0, CH)], sic.at[1 - p],
                            isem).wait()
                        pltpu.make_async_copy(
                            dst_hbm.at[wid, pl.ds(0, CH)], dic.at[1 - p],
                            isem).wait()
                        pltpu.async_copy(
                            xs_hbm.at[sic.at[1 - p, 0]],
                            rows_v.at[1 - buf], gsem)
                        pltpu.sync_copy(rows_v.at[buf],
                                        agg_sp.at[dic.at[p, b]], add=True)
                        cb = pl.multiple_of(
                            jnp.minimum((c + 2) * CH, nwin - CH), CH)
                        pltpu.async_copy(
                            src_hbm.at[wid, pl.ds(cb, CH)], sic.at[p],
                            isem)
                        pltpu.async_copy(
                            dst_hbm.at[wid, pl.ds(cb, CH)], dic.at[p],
                            isem)
            return carry

        lax.fori_loop(0, nchunks // 2, chunk_body, 0)
        pltpu.make_async_copy(
            xs_hbm.at[sic.at[0, 0]], rows_v.at[0], gsem).wait()
        pltpu.make_async_copy(
            src_hbm.at[wid, pl.ds(0, CH)], sic.at[0], isem).wait()
        pltpu.make_async_copy(
            dst_hbm.at[wid, pl.ds(0, CH)], dic.at[0], isem).wait()
        plsc.subcore_barrier()
        for off in range(0, rows_pt, WIN):
            sz = min(WIN, rows_pt - off)
            pltpu.sync_copy(
                agg_sp.at[pl.ds(pl.multiple_of(rbase + off, 8), sz)],
                rows_v.at[0, pl.ds(0, sz)])
            pltpu.sync_copy(
                rows_v.at[0, pl.ds(0, sz)],
                out_hbm.at[pl.ds(pl.multiple_of(obase + off, 8), sz)])

    return agg_kernel


def _norm_xs_body(d0_ref, d1_ref, h_ref, norm_ref, xs_ref):
    deg = jnp.maximum(d0_ref[...] + d1_ref[...], 1.0)
    nrm = lax.rsqrt(deg)
    norm_ref[...] = nrm
    xs_ref[...] = h_ref[...] * nrm


def _layer_body(p0_ref, p1_ref, h_ref, nrm_ref, w_ref, g_ref, b_ref,
                out_ref, *, beta, bn_scale):
    nrm = nrm_ref[...]
    smoothed = (p0_ref[...] + p1_ref[...]) * nrm
    feat = (1.0 - ALPHA) * smoothed + ALPHA * h_ref[...]
    z = (1.0 - beta) * feat + beta * jnp.dot(
        feat, w_ref[...], preferred_element_type=jnp.float32)
    y = jnp.maximum(z * bn_scale * g_ref[...] + b_ref[...], 0.0)
    out_ref[...] = y * nrm


def _final_body(p0_ref, p1_ref, h_ref, nrm_ref, w_ref, out_ref, *, beta):
    smoothed = (p0_ref[...] + p1_ref[...]) * nrm_ref[...]
    feat = (1.0 - ALPHA) * smoothed + ALPHA * h_ref[...]
    out_ref[...] = (1.0 - beta) * feat + beta * jnp.dot(
        feat, w_ref[...], preferred_element_type=jnp.float32)


def kernel(graph, h, W1, W2, W3, gamma1, beta1, gamma2, beta2):
    n, d = h.shape
    e = graph.shape[1]

    n_pad = ((n + NS * 8 - 1) // (NS * 8)) * (NS * 8)
    if n_pad == n:
        n_pad += NS * 8
    pad_rows = n_pad - n
    rows_pt = n_pad // NS
    quantum = NW * WIN * 2 * CH
    e_pad = ((e + quantum - 1) // quantum) * quantum
    nwin = e_pad // (NW * WIN)

    src = graph[0].astype(jnp.int32)
    dst = graph[1].astype(jnp.int32)
    pad_idx = n + (jnp.arange(e_pad - e, dtype=jnp.int32) % pad_rows)
    src_w = jnp.concatenate([src, pad_idx]).reshape(NW, nwin, WIN)
    dst_w = jnp.concatenate([dst, pad_idx]).reshape(NW, nwin, WIN)
    h_pad = jnp.pad(h, ((0, pad_rows), (0, 0)))
    zeros_nd = jnp.zeros((WIN, d), jnp.float32)
    zeros_n = jnp.zeros((rows_pt,), jnp.float32)

    deg_kernel = _make_deg_kernel(n_pad, nwin, rows_pt)
    agg_kernel = _make_agg_kernel(n_pad, d, nwin, rows_pt)

    dparts = deg_kernel(dst_w, zeros_n)

    blk = n_pad // 8
    grid = (n_pad // blk,)
    row_spec = pl.BlockSpec((blk, d), lambda i: (i, 0))
    col_spec = pl.BlockSpec((blk, 1), lambda i: (i, 0))
    w_spec = pl.BlockSpec((d, d), lambda i: (0, 0))
    vec_spec = pl.BlockSpec((1, d), lambda i: (0, 0))

    norm2d, xs = pl.pallas_call(
        _norm_xs_body,
        grid=grid,
        in_specs=[col_spec, col_spec, row_spec],
        out_specs=[col_spec, row_spec],
        out_shape=[jax.ShapeDtypeStruct((n_pad, 1), jnp.float32),
                   jax.ShapeDtypeStruct((n_pad, d), jnp.float32)],
    )(dparts[:n_pad].reshape(n_pad, 1), dparts[n_pad:].reshape(n_pad, 1),
      h_pad)

    beta_l = [math.log(1.0 / l + 1.0) for l in (1, 2, 3)]
    bn_scale = 1.0 / math.sqrt(1.0 + BN_EPS)

    for li, (w, g, b) in enumerate(((W1, gamma1, beta1),
                                    (W2, gamma2, beta2))):
        parts = agg_kernel(src_w, dst_w, xs, zeros_nd)
        xs = pl.pallas_call(
            functools.partial(_layer_body, beta=beta_l[li],
                              bn_scale=bn_scale),
            grid=grid,
            in_specs=[row_spec, row_spec, row_spec, col_spec, w_spec,
                      vec_spec, vec_spec],
            out_specs=row_spec,
            out_shape=jax.ShapeDtypeStruct((n_pad, d), jnp.float32),
        )(parts[:n_pad], parts[n_pad:], h_pad, norm2d, w,
          g.reshape(1, d), b.reshape(1, d))

    parts = agg_kernel(src_w, dst_w, xs, zeros_nd)
    out = pl.pallas_call(
        functools.partial(_final_body, beta=beta_l[2]),
        grid=grid,
        in_specs=[row_spec, row_spec, row_spec, col_spec, w_spec],
        out_specs=row_spec,
        out_shape=jax.ShapeDtypeStruct((n_pad, d), jnp.float32),
    )(parts[:n_pad], parts[n_pad:], h_pad, norm2d, W3)

    return out[:n]


# speedup vs baseline: 10.4361x; 1.1433x over previous
import functools
import math

import jax
import jax.numpy as jnp
from jax import lax
from jax.experimental import pallas as pl
from jax.experimental.pallas import tpu as pltpu
from jax.experimental.pallas import tpu_sc as plsc

ALPHA = 0.2
BN_EPS = 1e-5

NC = 2
NS = 16
NW = NC * NS
WIN = 64
CH = 8
NB = 4


def _sc_mesh():
    return plsc.VectorSubcoreMesh(core_axis_name="c", subcore_axis_name="s")


def _make_deg_kernel(n_pad, nwin, rows_pt):
    @functools.partial(
        pl.kernel,
        out_type=jax.ShapeDtypeStruct((NC * n_pad,), jnp.float32),
        mesh=_sc_mesh(),
        scratch_types=[
            pltpu.VMEM((nwin, WIN), jnp.int32),
            pltpu.VMEM((WIN,), jnp.float32),
            pltpu.VMEM((rows_pt,), jnp.float32),
            pltpu.VMEM_SHARED((n_pad,), jnp.float32),
        ],
    )
    def deg_kernel(dst_hbm, zeros_hbm, out_hbm, idx_v, ones_v, bounce_v,
                   deg_sp):
        cid = lax.axis_index("c")
        sid = lax.axis_index("s")
        wid = cid * NS + sid
        rbase = pl.multiple_of(sid * rows_pt, 8)
        obase = pl.multiple_of(cid * n_pad + rbase, 8)
        pltpu.sync_copy(zeros_hbm, bounce_v)
        pltpu.sync_copy(bounce_v, deg_sp.at[pl.ds(rbase, rows_pt)])
        for i in range(WIN // 16):
            ones_v[pl.ds(i * 16, 16)] = jnp.ones((16,), jnp.float32)
        pltpu.sync_copy(dst_hbm.at[wid], idx_v)
        plsc.subcore_barrier()

        def body(j, carry):
            pltpu.sync_copy(ones_v, deg_sp.at[idx_v.at[j]], add=True)
            return carry

        lax.fori_loop(0, nwin, body, 0)
        plsc.subcore_barrier()
        pltpu.sync_copy(deg_sp.at[pl.ds(rbase, rows_pt)], bounce_v)
        pltpu.sync_copy(bounce_v, out_hbm.at[pl.ds(obase, rows_pt)])

    return deg_kernel


def _make_agg_kernel(n_pad, d, nwin, rows_pt):
    @functools.partial(
        pl.kernel,
        out_type=jax.ShapeDtypeStruct((NC * n_pad, d), jnp.float32),
        mesh=_sc_mesh(),
        scratch_types=[
            pltpu.VMEM((2, CH, WIN), jnp.int32),
            pltpu.VMEM((2, CH, WIN), jnp.int32),
            pltpu.VMEM((NB, WIN, d), jnp.float32),
            pltpu.VMEM_SHARED((n_pad, d), jnp.float32),
            pltpu.SemaphoreType.DMA,
            pltpu.SemaphoreType.DMA,
            pltpu.SemaphoreType.DMA,
        ],
    )
    def agg_kernel(src_hbm, dst_hbm, xs_hbm, zeros_hbm, out_hbm,
                   sic, dic, rows_v, agg_sp, gsem, ssem, isem):
        nchunks = nwin // CH
        zb = NB - 1
        cid = lax.axis_index("c")
        sid = lax.axis_index("s")
        wid = cid * NS + sid
        rbase = sid * rows_pt
        obase = cid * n_pad + rbase
        row_chunks = []
        off = 0
        while off < rows_pt:
            row_chunks.append((off, min(WIN, rows_pt - off)))
            off += WIN
        pltpu.sync_copy(zeros_hbm, rows_v.at[zb])
        for off, sz in row_chunks:
            pltpu.async_copy(
                rows_v.at[zb, pl.ds(0, sz)],
                agg_sp.at[pl.ds(pl.multiple_of(rbase + off, 8), sz)],
                ssem)
        pltpu.sync_copy(src_hbm.at[wid, pl.ds(0, CH)], sic.at[0])
        pltpu.sync_copy(dst_hbm.at[wid, pl.ds(0, CH)], dic.at[0])
        for off, sz in row_chunks:
            pltpu.make_async_copy(
                rows_v.at[zb, pl.ds(0, sz)],
                agg_sp.at[pl.ds(pl.multiple_of(rbase + off, 8), sz)],
                ssem).wait()
        plsc.subcore_barrier()

        pltpu.async_copy(rows_v.at[zb], agg_sp.at[dic.at[0, 0]], ssem,
                         add=True)
        pltpu.async_copy(rows_v.at[zb], agg_sp.at[dic.at[0, 0]], ssem,
                         add=True)
        pltpu.async_copy(xs_hbm.at[sic.at[0, 0]], rows_v.at[0], gsem)
        pltpu.async_copy(xs_hbm.at[sic.at[0, 1]], rows_v.at[1], gsem)

        def chunk_body(c2, carry):
            for p in range(2):
                c = c2 * 2 + p
                for b in range(CH):
                    buf = b % NB
                    nbuf = (b + 2) % NB
                    pltpu.make_async_copy(
                        xs_hbm.at[sic.at[p, b]], rows_v.at[buf],
                        gsem).wait()
                    if b == 2:
                        cb = pl.multiple_of(
                            jnp.minimum((c + 1) * CH, nwin - CH), CH)
                        pltpu.async_copy(
                            src_hbm.at[wid, pl.ds(cb, CH)], sic.at[1 - p],
                            isem)
                        pltpu.async_copy(
                            dst_hbm.at[wid, pl.ds(cb, CH)], dic.at[1 - p],
                            isem)
                    if b == CH - 2:
                        pltpu.make_async_copy(
                            src_hbm.at[wid, pl.ds(0, CH)], sic.at[1 - p],
                            isem).wait()
                        pltpu.make_async_copy(
                            dst_hbm.at[wid, pl.ds(0, CH)], dic.at[1 - p],
                            isem).wait()
                    pltpu.make_async_copy(
                        rows_v.at[nbuf], agg_sp.at[dic.at[p, b]],
                        ssem).wait()
                    if b < CH - 2:
                        pltpu.async_copy(
                            xs_hbm.at[sic.at[p, b + 2]], rows_v.at[nbuf],
                            gsem)
                    else:
                        pltpu.async_copy(
                            xs_hbm.at[sic.at[1 - p, b + 2 - CH]],
                            rows_v.at[nbuf], gsem)
                    pltpu.async_copy(rows_v.at[buf],
                                     agg_sp.at[dic.at[p, b]], ssem,
                                     add=True)
            return carry

        lax.fori_loop(0, nchunks // 2, chunk_body, 0)
        pltpu.make_async_copy(
            xs_hbm.at[sic.at[0, 0]], rows_v.at[0], gsem).wait()
        pltpu.make_async_copy(
            xs_hbm.at[sic.at[0, 1]], rows_v.at[1], gsem).wait()
        pltpu.make_async_copy(
            rows_v.at[2], agg_sp.at[dic.at[0, 0]], ssem).wait()
        pltpu.make_async_copy(
            rows_v.at[3], agg_sp.at[dic.at[0, 0]], ssem).wait()
        plsc.subcore_barrier()
        for k, (off, sz) in enumerate(row_chunks):
            buf = k % 2
            if k >= 2:
                poff, psz = row_chunks[k - 2]
                pltpu.make_async_copy(
                    rows_v.at[k % 2, pl.ds(0, psz)],
                    out_hbm.at[pl.ds(pl.multiple_of(obase + poff, 8),
                                     psz)],
                    gsem).wait()
            pltpu.sync_copy(
                agg_sp.at[pl.ds(pl.multiple_of(rbase + off, 8), sz)],
                rows_v.at[buf, pl.ds(0, sz)])
            pltpu.async_copy(
                rows_v.at[buf, pl.ds(0, sz)],
                out_hbm.at[pl.ds(pl.multiple_of(obase + off, 8), sz)],
                gsem)
        for k in (len(row_chunks) - 2, len(row_chunks) - 1):
            off, sz = row_chunks[k]
            pltpu.make_async_copy(
                rows_v.at[k % 2, pl.ds(0, sz)],
                out_hbm.at[pl.ds(pl.multiple_of(obase + off, 8), sz)],
                gsem).wait()

    return agg_kernel


def _norm_xs_body(d0_ref, d1_ref, h_ref, norm_ref, xs_ref):
    deg = jnp.maximum(d0_ref[...] + d1_ref[...], 1.0)
    nrm = lax.rsqrt(deg)
    norm_ref[...] = nrm
    xs_ref[...] = h_ref[...] * nrm


def _layer_body(p0_ref, p1_ref, h_ref, nrm_ref, w_ref, g_ref, b_ref,
                out_ref, *, beta, bn_scale):
    nrm = nrm_ref[...]
    smoothed = (p0_ref[...] + p1_ref[...]) * nrm
    feat = (1.0 - ALPHA) * smoothed + ALPHA * h_ref[...]
    z = (1.0 - beta) * feat + beta * jnp.dot(
        feat, w_ref[...], preferred_element_type=jnp.float32)
    y = jnp.maximum(z * bn_scale * g_ref[...] + b_ref[...], 0.0)
    out_ref[...] = y * nrm


def _final_body(p0_ref, p1_ref, h_ref, nrm_ref, w_ref, out_ref, *, beta):
    smoothed = (p0_ref[...] + p1_ref[...]) * nrm_ref[...]
    feat = (1.0 - ALPHA) * smoothed + ALPHA * h_ref[...]
    out_ref[...] = (1.0 - beta) * feat + beta * jnp.dot(
        feat, w_ref[...], preferred_element_type=jnp.float32)


def kernel(graph, h, W1, W2, W3, gamma1, beta1, gamma2, beta2):
    n, d = h.shape
    e = graph.shape[1]

    n_pad = ((n + NS * 8 - 1) // (NS * 8)) * (NS * 8)
    if n_pad == n:
        n_pad += NS * 8
    pad_rows = n_pad - n
    rows_pt = n_pad // NS
    quantum = NW * WIN * 2 * CH
    e_pad = ((e + quantum - 1) // quantum) * quantum
    nwin = e_pad // (NW * WIN)

    src = graph[0].astype(jnp.int32)
    dst = graph[1].astype(jnp.int32)
    pad_idx = n + (jnp.arange(e_pad - e, dtype=jnp.int32) % pad_rows)
    src_w = jnp.concatenate([src, pad_idx]).reshape(NW, nwin, WIN)
    dst_w = jnp.concatenate([dst, pad_idx]).reshape(NW, nwin, WIN)
    h_pad = jnp.pad(h, ((0, pad_rows), (0, 0)))
    zeros_nd = jnp.zeros((WIN, d), jnp.float32)
    zeros_n = jnp.zeros((rows_pt,), jnp.float32)

    deg_kernel = _make_deg_kernel(n_pad, nwin, rows_pt)
    agg_kernel = _make_agg_kernel(n_pad, d, nwin, rows_pt)

    dparts = deg_kernel(dst_w, zeros_n)

    blk = n_pad // 8
    grid = (n_pad // blk,)
    row_spec = pl.BlockSpec((blk, d), lambda i: (i, 0))
    col_spec = pl.BlockSpec((blk, 1), lambda i: (i, 0))
    w_spec = pl.BlockSpec((d, d), lambda i: (0, 0))
    vec_spec = pl.BlockSpec((1, d), lambda i: (0, 0))

    norm2d, xs = pl.pallas_call(
        _norm_xs_body,
        grid=grid,
        in_specs=[col_spec, col_spec, row_spec],
        out_specs=[col_spec, row_spec],
        out_shape=[jax.ShapeDtypeStruct((n_pad, 1), jnp.float32),
                   jax.ShapeDtypeStruct((n_pad, d), jnp.float32)],
    )(dparts[:n_pad].reshape(n_pad, 1), dparts[n_pad:].reshape(n_pad, 1),
      h_pad)

    beta_l = [math.log(1.0 / l + 1.0) for l in (1, 2, 3)]
    bn_scale = 1.0 / math.sqrt(1.0 + BN_EPS)

    for li, (w, g, b) in enumerate(((W1, gamma1, beta1),
                                    (W2, gamma2, beta2))):
        parts = agg_kernel(src_w, dst_w, xs, zeros_nd)
        xs = pl.pallas_call(
            functools.partial(_layer_body, beta=beta_l[li],
                              bn_scale=bn_scale),
            grid=grid,
            in_specs=[row_spec, row_spec, row_spec, col_spec, w_spec,
                      vec_spec, vec_spec],
            out_specs=row_spec,
            out_shape=jax.ShapeDtypeStruct((n_pad, d), jnp.float32),
        )(parts[:n_pad], parts[n_pad:], h_pad, norm2d, w,
          g.reshape(1, d), b.reshape(1, d))

    parts = agg_kernel(src_w, dst_w, xs, zeros_nd)
    out = pl.pallas_call(
        functools.partial(_final_body, beta=beta_l[2]),
        grid=grid,
        in_specs=[row_spec, row_spec, row_spec, col_spec, w_spec],
        out_specs=row_spec,
        out_shape=jax.ShapeDtypeStruct((n_pad, d), jnp.float32),
    )(parts[:n_pad], parts[n_pad:], h_pad, norm2d, W3)

    return out[:n]


# speedup vs baseline: 10.6795x; 1.0233x over previous
import functools
import math

import jax
import jax.numpy as jnp
from jax import lax
from jax.experimental import pallas as pl
from jax.experimental.pallas import tpu as pltpu
from jax.experimental.pallas import tpu_sc as plsc

ALPHA = 0.2
BN_EPS = 1e-5

NC = 2
NS = 16
NW = NC * NS
WIN = 64
CH = 8
NB = 4


def _sc_mesh():
    return plsc.VectorSubcoreMesh(core_axis_name="c", subcore_axis_name="s")


def _make_deg_kernel(n_pad, dwin, rows_pt):
    DW = 128

    @functools.partial(
        pl.kernel,
        out_type=jax.ShapeDtypeStruct((NC * n_pad,), jnp.float32),
        mesh=_sc_mesh(),
        scratch_types=[
            pltpu.VMEM((dwin, DW), jnp.int32),
            pltpu.VMEM((DW,), jnp.float32),
            pltpu.VMEM((rows_pt,), jnp.float32),
            pltpu.VMEM_SHARED((n_pad,), jnp.float32),
            pltpu.SemaphoreType.DMA,
        ],
    )
    def deg_kernel(dst_hbm, zeros_hbm, out_hbm, idx_v, ones_v, bounce_v,
                   deg_sp, sem):
        cid = lax.axis_index("c")
        sid = lax.axis_index("s")
        wid = cid * NS + sid
        rbase = pl.multiple_of(sid * rows_pt, 8)
        obase = pl.multiple_of(cid * n_pad + rbase, 8)
        pltpu.sync_copy(zeros_hbm, bounce_v)
        pltpu.sync_copy(bounce_v, deg_sp.at[pl.ds(rbase, rows_pt)])
        for i in range(DW // 16):
            ones_v[pl.ds(i * 16, 16)] = jnp.ones((16,), jnp.float32)
        pltpu.sync_copy(dst_hbm.at[wid], idx_v)
        plsc.subcore_barrier()

        def body(j, carry):
            pltpu.async_copy(ones_v, deg_sp.at[idx_v.at[j]], sem,
                             add=True)
            return carry

        lax.fori_loop(0, dwin, body, 0)

        def drain(j, carry):
            pltpu.make_async_copy(ones_v, deg_sp.at[idx_v.at[0]],
                                  sem).wait()
            return carry

        lax.fori_loop(0, dwin, drain, 0)
        plsc.subcore_barrier()
        pltpu.sync_copy(deg_sp.at[pl.ds(rbase, rows_pt)], bounce_v)
        pltpu.sync_copy(bounce_v, out_hbm.at[pl.ds(obase, rows_pt)])

    return deg_kernel


def _make_agg_kernel(n_pad, d, nwin, rows_pt):
    @functools.partial(
        pl.kernel,
        out_type=jax.ShapeDtypeStruct((NC * n_pad, d), jnp.float32),
        mesh=_sc_mesh(),
        scratch_types=[
            pltpu.VMEM((2, CH, WIN), jnp.int32),
            pltpu.VMEM((2, CH, WIN), jnp.int32),
            pltpu.VMEM((NB, WIN, d), jnp.float32),
            pltpu.VMEM_SHARED((n_pad, d), jnp.float32),
            pltpu.SemaphoreType.DMA,
            pltpu.SemaphoreType.DMA,
            pltpu.SemaphoreType.DMA,
        ],
    )
    def agg_kernel(src_hbm, dst_hbm, xs_hbm, zeros_hbm, out_hbm,
                   sic, dic, rows_v, agg_sp, gsem, ssem, isem):
        nchunks = nwin // CH
        zb = NB - 1
        cid = lax.axis_index("c")
        sid = lax.axis_index("s")
        wid = cid * NS + sid
        rbase = sid * rows_pt
        obase = cid * n_pad + rbase
        row_chunks = []
        off = 0
        while off < rows_pt:
            row_chunks.append((off, min(WIN, rows_pt - off)))
            off += WIN
        pltpu.sync_copy(zeros_hbm, rows_v.at[zb])
        for off, sz in row_chunks:
            pltpu.async_copy(
                rows_v.at[zb, pl.ds(0, sz)],
                agg_sp.at[pl.ds(pl.multiple_of(rbase + off, 8), sz)],
                ssem)
        pltpu.sync_copy(src_hbm.at[wid, pl.ds(0, CH)], sic.at[0])
        pltpu.sync_copy(dst_hbm.at[wid, pl.ds(0, CH)], dic.at[0])
        for off, sz in row_chunks:
            pltpu.make_async_copy(
                rows_v.at[zb, pl.ds(0, sz)],
                agg_sp.at[pl.ds(pl.multiple_of(rbase + off, 8), sz)],
                ssem).wait()
        plsc.subcore_barrier()

        pltpu.async_copy(rows_v.at[zb], agg_sp.at[dic.at[0, 0]], ssem,
                         add=True)
        pltpu.async_copy(rows_v.at[zb], agg_sp.at[dic.at[0, 0]], ssem,
                         add=True)
        pltpu.async_copy(xs_hbm.at[sic.at[0, 0]], rows_v.at[0], gsem)
        pltpu.async_copy(xs_hbm.at[sic.at[0, 1]], rows_v.at[1], gsem)

        def chunk_body(c2, carry):
            for p in range(2):
                c = c2 * 2 + p
                for b in range(CH):
                    buf = b % NB
                    nbuf = (b + 2) % NB
                    pltpu.make_async_copy(
                        xs_hbm.at[sic.at[p, b]], rows_v.at[buf],
                        gsem).wait()
                    if b == 2:
                        cb = pl.multiple_of(
                            jnp.minimum((c + 1) * CH, nwin - CH), CH)
                        pltpu.async_copy(
                            src_hbm.at[wid, pl.ds(cb, CH)], sic.at[1 - p],
                            isem)
                        pltpu.async_copy(
                            dst_hbm.at[wid, pl.ds(cb, CH)], dic.at[1 - p],
                            isem)
                    if b == CH - 2:
                        pltpu.make_async_copy(
                            src_hbm.at[wid, pl.ds(0, CH)], sic.at[1 - p],
                            isem).wait()
                        pltpu.make_async_copy(
                            dst_hbm.at[wid, pl.ds(0, CH)], dic.at[1 - p],
                            isem).wait()
                    pltpu.make_async_copy(
                        rows_v.at[nbuf], agg_sp.at[dic.at[p, b]],
                        ssem).wait()
                    if b < CH - 2:
                        pltpu.async_copy(
                            xs_hbm.at[sic.at[p, b + 2]], rows_v.at[nbuf],
                            gsem)
                    else:
                        pltpu.async_copy(
                            xs_hbm.at[sic.at[1 - p, b + 2 - CH]],
                            rows_v.at[nbuf], gsem)
                    pltpu.async_copy(rows_v.at[buf],
                                     agg_sp.at[dic.at[p, b]], ssem,
                                     add=True)
            return carry

        lax.fori_loop(0, nchunks // 2, chunk_body, 0)
        pltpu.make_async_copy(
            xs_hbm.at[sic.at[0, 0]], rows_v.at[0], gsem).wait()
        pltpu.make_async_copy(
            xs_hbm.at[sic.at[0, 1]], rows_v.at[1], gsem).wait()
        pltpu.make_async_copy(
            rows_v.at[2], agg_sp.at[dic.at[0, 0]], ssem).wait()
        pltpu.make_async_copy(
            rows_v.at[3], agg_sp.at[dic.at[0, 0]], ssem).wait()
        plsc.subcore_barrier()
        for k, (off, sz) in enumerate(row_chunks):
            buf = k % 2
            if k >= 2:
                poff, psz = row_chunks[k - 2]
                pltpu.make_async_copy(
                    rows_v.at[k % 2, pl.ds(0, psz)],
                    out_hbm.at[pl.ds(pl.multiple_of(obase + poff, 8),
                                     psz)],
                    gsem).wait()
            pltpu.sync_copy(
                agg_sp.at[pl.ds(pl.multiple_of(rbase + off, 8), sz)],
                rows_v.at[buf, pl.ds(0, sz)])
            pltpu.async_copy(
                rows_v.at[buf, pl.ds(0, sz)],
                out_hbm.at[pl.ds(pl.multiple_of(obase + off, 8), sz)],
                gsem)
        for k in (len(row_chunks) - 2, len(row_chunks) - 1):
            off, sz = row_chunks[k]
            pltpu.make_async_copy(
                rows_v.at[k % 2, pl.ds(0, sz)],
                out_hbm.at[pl.ds(pl.multiple_of(obase + off, 8), sz)],
                gsem).wait()

    return agg_kernel


def _norm_xs_body(d0_ref, d1_ref, h_ref, norm_ref, xs_ref):
    deg = jnp.maximum(d0_ref[...] + d1_ref[...], 1.0)
    nrm = lax.rsqrt(deg)
    norm_ref[...] = nrm
    xs_ref[...] = h_ref[...] * nrm


def _layer_body(p0_ref, p1_ref, h_ref, nrm_ref, w_ref, g_ref, b_ref,
                out_ref, *, beta, bn_scale):
    nrm = nrm_ref[...]
    smoothed = (p0_ref[...] + p1_ref[...]) * nrm
    feat = (1.0 - ALPHA) * smoothed + ALPHA * h_ref[...]
    z = (1.0 - beta) * feat + beta * jnp.dot(
        feat, w_ref[...], preferred_element_type=jnp.float32)
    y = jnp.maximum(z * bn_scale * g_ref[...] + b_ref[...], 0.0)
    out_ref[...] = y * nrm


def _final_body(p0_ref, p1_ref, h_ref, nrm_ref, w_ref, out_ref, *, beta):
    smoothed = (p0_ref[...] + p1_ref[...]) * nrm_ref[...]
    feat = (1.0 - ALPHA) * smoothed + ALPHA * h_ref[...]
    out_ref[...] = (1.0 - beta) * feat + beta * jnp.dot(
        feat, w_ref[...], preferred_element_type=jnp.float32)


def kernel(graph, h, W1, W2, W3, gamma1, beta1, gamma2, beta2):
    n, d = h.shape
    e = graph.shape[1]

    n_pad = ((n + NS * 8 - 1) // (NS * 8)) * (NS * 8)
    if n_pad == n:
        n_pad += NS * 8
    pad_rows = n_pad - n
    rows_pt = n_pad // NS
    quantum = NW * WIN * 2 * CH
    e_pad = ((e + quantum - 1) // quantum) * quantum
    nwin = e_pad // (NW * WIN)

    src = graph[0].astype(jnp.int32)
    dst = graph[1].astype(jnp.int32)
    pad_idx = n + (jnp.arange(e_pad - e, dtype=jnp.int32) % pad_rows)
    src_w = jnp.concatenate([src, pad_idx]).reshape(NW, nwin, WIN)
    dst_w = jnp.concatenate([dst, pad_idx]).reshape(NW, nwin, WIN)
    h_pad = jnp.pad(h, ((0, pad_rows), (0, 0)))
    zeros_nd = jnp.zeros((WIN, d), jnp.float32)
    zeros_n = jnp.zeros((rows_pt,), jnp.float32)

    dwin = e_pad // (NW * 128)
    deg_kernel = _make_deg_kernel(n_pad, dwin, rows_pt)
    agg_kernel = _make_agg_kernel(n_pad, d, nwin, rows_pt)

    dparts = deg_kernel(dst_w.reshape(NW, dwin, 128), zeros_n)

    blk = n_pad // 8
    grid = (n_pad // blk,)
    row_spec = pl.BlockSpec((blk, d), lambda i: (i, 0))
    col_spec = pl.BlockSpec((blk, 1), lambda i: (i, 0))
    w_spec = pl.BlockSpec((d, d), lambda i: (0, 0))
    vec_spec = pl.BlockSpec((1, d), lambda i: (0, 0))

    norm2d, xs = pl.pallas_call(
        _norm_xs_body,
        grid=grid,
        in_specs=[col_spec, col_spec, row_spec],
        out_specs=[col_spec, row_spec],
        out_shape=[jax.ShapeDtypeStruct((n_pad, 1), jnp.float32),
                   jax.ShapeDtypeStruct((n_pad, d), jnp.float32)],
    )(dparts[:n_pad].reshape(n_pad, 1), dparts[n_pad:].reshape(n_pad, 1),
      h_pad)

    beta_l = [math.log(1.0 / l + 1.0) for l in (1, 2, 3)]
    bn_scale = 1.0 / math.sqrt(1.0 + BN_EPS)

    for li, (w, g, b) in enumerate(((W1, gamma1, beta1),
                                    (W2, gamma2, beta2))):
        parts = agg_kernel(src_w, dst_w, xs, zeros_nd)
        xs = pl.pallas_call(
            functools.partial(_layer_body, beta=beta_l[li],
                              bn_scale=bn_scale),
            grid=grid,
            in_specs=[row_spec, row_spec, row_spec, col_spec, w_spec,
                      vec_spec, vec_spec],
            out_specs=row_spec,
            out_shape=jax.ShapeDtypeStruct((n_pad, d), jnp.float32),
        )(parts[:n_pad], parts[n_pad:], h_pad, norm2d, w,
          g.reshape(1, d), b.reshape(1, d))

    parts = agg_kernel(src_w, dst_w, xs, zeros_nd)
    out = pl.pallas_call(
        functools.partial(_final_body, beta=beta_l[2]),
        grid=grid,
        in_specs=[row_spec, row_spec, row_spec, col_spec, w_spec],
        out_specs=row_spec,
        out_shape=jax.ShapeDtypeStruct((n_pad, d), jnp.float32),
    )(parts[:n_pad], parts[n_pad:], h_pad, norm2d, W3)

    return out[:n]


# speedup vs baseline: 11.1141x; 1.0407x over previous
import functools
import math

import jax
import jax.numpy as jnp
from jax import lax
from jax.experimental import pallas as pl
from jax.experimental.pallas import tpu as pltpu
from jax.experimental.pallas import tpu_sc as plsc

ALPHA = 0.2
BN_EPS = 1e-5

NC = 2
NS = 16
NW = NC * NS
WIN = 80
CH = 8
NB = 4


def _sc_mesh():
    return plsc.VectorSubcoreMesh(core_axis_name="c", subcore_axis_name="s")


def _make_deg_kernel(n_pad, dwin, rows_pt):
    DW = 128

    @functools.partial(
        pl.kernel,
        out_type=jax.ShapeDtypeStruct((NC * n_pad,), jnp.float32),
        mesh=_sc_mesh(),
        scratch_types=[
            pltpu.VMEM((dwin, DW), jnp.int32),
            pltpu.VMEM((DW,), jnp.float32),
            pltpu.VMEM((rows_pt,), jnp.float32),
            pltpu.VMEM_SHARED((n_pad,), jnp.float32),
            pltpu.SemaphoreType.DMA,
        ],
    )
    def deg_kernel(dst_hbm, zeros_hbm, out_hbm, idx_v, ones_v, bounce_v,
                   deg_sp, sem):
        cid = lax.axis_index("c")
        sid = lax.axis_index("s")
        wid = cid * NS + sid
        rbase = pl.multiple_of(sid * rows_pt, 8)
        obase = pl.multiple_of(cid * n_pad + rbase, 8)
        pltpu.sync_copy(zeros_hbm, bounce_v)
        pltpu.sync_copy(bounce_v, deg_sp.at[pl.ds(rbase, rows_pt)])
        for i in range(DW // 16):
            ones_v[pl.ds(i * 16, 16)] = jnp.ones((16,), jnp.float32)
        pltpu.sync_copy(dst_hbm.at[wid], idx_v)
        plsc.subcore_barrier()

        def body(j, carry):
            pltpu.async_copy(ones_v, deg_sp.at[idx_v.at[j]], sem,
                             add=True)
            return carry

        lax.fori_loop(0, dwin, body, 0)

        def drain(j, carry):
            pltpu.make_async_copy(ones_v, deg_sp.at[idx_v.at[0]],
                                  sem).wait()
            return carry

        lax.fori_loop(0, dwin, drain, 0)
        plsc.subcore_barrier()
        pltpu.sync_copy(deg_sp.at[pl.ds(rbase, rows_pt)], bounce_v)
        pltpu.sync_copy(bounce_v, out_hbm.at[pl.ds(obase, rows_pt)])

    return deg_kernel


def _make_agg_kernel(n_pad, d, nwin, rows_pt):
    @functools.partial(
        pl.kernel,
        out_type=jax.ShapeDtypeStruct((NC * n_pad, d), jnp.float32),
        mesh=_sc_mesh(),
        scratch_types=[
            pltpu.VMEM((2, CH, WIN), jnp.int32),
            pltpu.VMEM((2, CH, WIN), jnp.int32),
            pltpu.VMEM((NB, WIN, d), jnp.float32),
            pltpu.VMEM_SHARED((n_pad, d), jnp.float32),
            pltpu.SemaphoreType.DMA,
            pltpu.SemaphoreType.DMA,
            pltpu.SemaphoreType.DMA,
        ],
    )
    def agg_kernel(src_hbm, dst_hbm, xs_hbm, zeros_hbm, out_hbm,
                   sic, dic, rows_v, agg_sp, gsem, ssem, isem):
        nchunks = nwin // CH
        zb = NB - 1
        cid = lax.axis_index("c")
        sid = lax.axis_index("s")
        wid = cid * NS + sid
        rbase = sid * rows_pt
        obase = cid * n_pad + rbase
        row_chunks = []
        off = 0
        while off < rows_pt:
            row_chunks.append((off, min(WIN, rows_pt - off)))
            off += WIN
        pltpu.sync_copy(zeros_hbm, rows_v.at[zb])
        for off, sz in row_chunks:
            pltpu.async_copy(
                rows_v.at[zb, pl.ds(0, sz)],
                agg_sp.at[pl.ds(pl.multiple_of(rbase + off, 8), sz)],
                ssem)
        pltpu.sync_copy(src_hbm.at[wid, pl.ds(0, CH)], sic.at[0])
        pltpu.sync_copy(dst_hbm.at[wid, pl.ds(0, CH)], dic.at[0])
        for off, sz in row_chunks:
            pltpu.make_async_copy(
                rows_v.at[zb, pl.ds(0, sz)],
                agg_sp.at[pl.ds(pl.multiple_of(rbase + off, 8), sz)],
                ssem).wait()
        plsc.subcore_barrier()

        pltpu.async_copy(rows_v.at[zb], agg_sp.at[dic.at[0, 0]], ssem,
                         add=True)
        pltpu.async_copy(rows_v.at[zb], agg_sp.at[dic.at[0, 0]], ssem,
                         add=True)
        pltpu.async_copy(xs_hbm.at[sic.at[0, 0]], rows_v.at[0], gsem)
        pltpu.async_copy(xs_hbm.at[sic.at[0, 1]], rows_v.at[1], gsem)

        def chunk_body(c2, carry):
            for p in range(2):
                c = c2 * 2 + p
                for b in range(CH):
                    buf = b % NB
                    nbuf = (b + 2) % NB
                    pltpu.make_async_copy(
                        xs_hbm.at[sic.at[p, b]], rows_v.at[buf],
                        gsem).wait()
                    if b == 2:
                        cb = pl.multiple_of(
                            jnp.minimum((c + 1) * CH, nwin - CH), CH)
                        pltpu.async_copy(
                            src_hbm.at[wid, pl.ds(cb, CH)], sic.at[1 - p],
                            isem)
                        pltpu.async_copy(
                            dst_hbm.at[wid, pl.ds(cb, CH)], dic.at[1 - p],
                            isem)
                    if b == CH - 2:
                        pltpu.make_async_copy(
                            src_hbm.at[wid, pl.ds(0, CH)], sic.at[1 - p],
                            isem).wait()
                        pltpu.make_async_copy(
                            dst_hbm.at[wid, pl.ds(0, CH)], dic.at[1 - p],
                            isem).wait()
                    pltpu.make_async_copy(
                        rows_v.at[nbuf], agg_sp.at[dic.at[p, b]],
                        ssem).wait()
                    if b < CH - 2:
                        pltpu.async_copy(
                            xs_hbm.at[sic.at[p, b + 2]], rows_v.at[nbuf],
                            gsem)
                    else:
                        pltpu.async_copy(
                            xs_hbm.at[sic.at[1 - p, b + 2 - CH]],
                            rows_v.at[nbuf], gsem)
                    pltpu.async_copy(rows_v.at[buf],
                                     agg_sp.at[dic.at[p, b]], ssem,
                                     add=True)
            return carry

        lax.fori_loop(0, nchunks // 2, chunk_body, 0)
        pltpu.make_async_copy(
            xs_hbm.at[sic.at[0, 0]], rows_v.at[0], gsem).wait()
        pltpu.make_async_copy(
            xs_hbm.at[sic.at[0, 1]], rows_v.at[1], gsem).wait()
        pltpu.make_async_copy(
            rows_v.at[2], agg_sp.at[dic.at[0, 0]], ssem).wait()
        pltpu.make_async_copy(
            rows_v.at[3], agg_sp.at[dic.at[0, 0]], ssem).wait()
        plsc.subcore_barrier()
        for k, (off, sz) in enumerate(row_chunks):
            buf = k % 2
            if k >= 2:
                poff, psz = row_chunks[k - 2]
                pltpu.make_async_copy(
                    rows_v.at[k % 2, pl.ds(0, psz)],
                    out_hbm.at[pl.ds(pl.multiple_of(obase + poff, 8),
                                     psz)],
                    gsem).wait()
            pltpu.sync_copy(
                agg_sp.at[pl.ds(pl.multiple_of(rbase + off, 8), sz)],
                rows_v.at[buf, pl.ds(0, sz)])
            pltpu.async_copy(
                rows_v.at[buf, pl.ds(0, sz)],
                out_hbm.at[pl.ds(pl.multiple_of(obase + off, 8), sz)],
                gsem)
        for k in (len(row_chunks) - 2, len(row_chunks) - 1):
            off, sz = row_chunks[k]
            pltpu.make_async_copy(
                rows_v.at[k % 2, pl.ds(0, sz)],
                out_hbm.at[pl.ds(pl.multiple_of(obase + off, 8), sz)],
                gsem).wait()

    return agg_kernel


def _norm_xs_body(d0_ref, d1_ref, h_ref, norm_ref, xs_ref):
    deg = jnp.maximum(d0_ref[...] + d1_ref[...], 1.0)
    nrm = lax.rsqrt(deg)
    norm_ref[...] = nrm
    xs_ref[...] = h_ref[...] * nrm


def _layer_body(p0_ref, p1_ref, h_ref, nrm_ref, w_ref, g_ref, b_ref,
                out_ref, *, beta, bn_scale):
    nrm = nrm_ref[...]
    smoothed = (p0_ref[...] + p1_ref[...]) * nrm
    feat = (1.0 - ALPHA) * smoothed + ALPHA * h_ref[...]
    z = (1.0 - beta) * feat + beta * jnp.dot(
        feat, w_ref[...], preferred_element_type=jnp.float32)
    y = jnp.maximum(z * bn_scale * g_ref[...] + b_ref[...], 0.0)
    out_ref[...] = y * nrm


def _final_body(p0_ref, p1_ref, h_ref, nrm_ref, w_ref, out_ref, *, beta):
    smoothed = (p0_ref[...] + p1_ref[...]) * nrm_ref[...]
    feat = (1.0 - ALPHA) * smoothed + ALPHA * h_ref[...]
    out_ref[...] = (1.0 - beta) * feat + beta * jnp.dot(
        feat, w_ref[...], preferred_element_type=jnp.float32)


def kernel(graph, h, W1, W2, W3, gamma1, beta1, gamma2, beta2):
    n, d = h.shape
    e = graph.shape[1]

    n_pad = ((n + NS * 8 - 1) // (NS * 8)) * (NS * 8)
    if n_pad == n:
        n_pad += NS * 8
    pad_rows = n_pad - n
    rows_pt = n_pad // NS
    quantum = NW * WIN * 2 * CH
    e_pad = ((e + quantum - 1) // quantum) * quantum
    nwin = e_pad // (NW * WIN)

    src = graph[0].astype(jnp.int32)
    dst = graph[1].astype(jnp.int32)
    pad_idx = n + (jnp.arange(e_pad - e, dtype=jnp.int32) % pad_rows)
    src_w = jnp.concatenate([src, pad_idx]).reshape(NW, nwin, WIN)
    dst_w = jnp.concatenate([dst, pad_idx]).reshape(NW, nwin, WIN)
    h_pad = jnp.pad(h, ((0, pad_rows), (0, 0)))
    zeros_nd = jnp.zeros((WIN, d), jnp.float32)
    zeros_n = jnp.zeros((rows_pt,), jnp.float32)

    dwin = e_pad // (NW * 128)
    deg_kernel = _make_deg_kernel(n_pad, dwin, rows_pt)
    agg_kernel = _make_agg_kernel(n_pad, d, nwin, rows_pt)

    dparts = deg_kernel(dst_w.reshape(NW, dwin, 128), zeros_n)

    blk = n_pad // 8
    grid = (n_pad // blk,)
    row_spec = pl.BlockSpec((blk, d), lambda i: (i, 0))
    col_spec = pl.BlockSpec((blk, 1), lambda i: (i, 0))
    w_spec = pl.BlockSpec((d, d), lambda i: (0, 0))
    vec_spec = pl.BlockSpec((1, d), lambda i: (0, 0))

    norm2d, xs = pl.pallas_call(
        _norm_xs_body,
        grid=grid,
        in_specs=[col_spec, col_spec, row_spec],
        out_specs=[col_spec, row_spec],
        out_shape=[jax.ShapeDtypeStruct((n_pad, 1), jnp.float32),
                   jax.ShapeDtypeStruct((n_pad, d), jnp.float32)],
    )(dparts[:n_pad].reshape(n_pad, 1), dparts[n_pad:].reshape(n_pad, 1),
      h_pad)

    beta_l = [math.log(1.0 / l + 1.0) for l in (1, 2, 3)]
    bn_scale = 1.0 / math.sqrt(1.0 + BN_EPS)

    for li, (w, g, b) in enumerate(((W1, gamma1, beta1),
                                    (W2, gamma2, beta2))):
        parts = agg_kernel(src_w, dst_w, xs, zeros_nd)
        xs = pl.pallas_call(
            functools.partial(_layer_body, beta=beta_l[li],
                              bn_scale=bn_scale),
            grid=grid,
            in_specs=[row_spec, row_spec, row_spec, col_spec, w_spec,
                      vec_spec, vec_spec],
            out_specs=row_spec,
            out_shape=jax.ShapeDtypeStruct((n_pad, d), jnp.float32),
        )(parts[:n_pad], parts[n_pad:], h_pad, norm2d, w,
          g.reshape(1, d), b.reshape(1, d))

    parts = agg_kernel(src_w, dst_w, xs, zeros_nd)
    out = pl.pallas_call(
        functools.partial(_final_body, beta=beta_l[2]),
        grid=grid,
        in_specs=[row_spec, row_spec, row_spec, col_spec, w_spec],
        out_specs=row_spec,
        out_shape=jax.ShapeDtypeStruct((n_pad, d), jnp.float32),
    )(parts[:n_pad], parts[n_pad:], h_pad, norm2d, W3)

    return out[:n]
